# baseline (speedup 1.0000x reference)
import functools
import math

import jax
import jax.numpy as jnp
from jax import lax
from jax.experimental import pallas as pl
from jax.experimental.pallas import tpu as pltpu

_SPIKE_THRESHOLD = math.log(0.9)
_B, _T, _V = 16, 2048, 512
_TCH = 512
_NCH = _T // _TCH


def _tc_body(tl_ref, alpha_ref, ctc_ref, mask_ref, out_ref, n_acc, rv_acc):
    b = pl.program_id(0)
    c = pl.program_id(1)

    @pl.when(jnp.logical_and(b == 0, c == 0))
    def _init():
        n_acc[...] = jnp.zeros_like(n_acc)
        rv_acc[...] = jnp.zeros_like(rv_acc)

    x = ctc_ref[0]
    blank = x[:, 0:1]
    trig = ((1.0 - blank) > _SPIKE_THRESHOLD).astype(jnp.float32)
    mrow = mask_ref[pl.ds(b, 1), pl.ds(c * _TCH, _TCH)]
    m01 = (mrow != 0.0).astype(jnp.float32)
    cnt = jnp.dot(m01, trig, preferred_element_type=jnp.float32)[0, 0]
    asum = jnp.sum(alpha_ref[pl.ds(b, 1), pl.ds(c * _TCH, _TCH)])

    lane = lax.broadcasted_iota(jnp.int32, (8, 128), 1)
    row = lax.broadcasted_iota(jnp.int32, (8, 128), 0)
    sel = jnp.logical_and(row == 0, lane == b)
    n_acc[...] += jnp.where(sel, cnt, 0.0)
    rv_acc[...] += jnp.where(sel, asum, 0.0)

    @pl.when(jnp.logical_and(b == _B - 1, c == _NCH - 1))
    def _final():
        lanes = lax.broadcasted_iota(jnp.int32, (1, 128), 1)
        counts = n_acc[0:1, :]
        rvs = rv_acc[0:1, :]
        lt = jnp.zeros((1, 128), jnp.float32)
        for i in range(_B):
            tli = tl_ref[i].astype(jnp.float32)
            lt += jnp.where(lanes == i, tli, 0.0)
        has = counts >= 1.0
        n = jnp.where(has, counts, 1.0)
        rv = jnp.where(has, rvs, 1.0)
        valid = lanes < _B
        max_s = jnp.max(jnp.where(valid, n, 0.0))
        max_len = jnp.maximum(1.0, jnp.max(lt))
        length = jnp.minimum(max_s, max_len)
        l_b = jnp.minimum(lt, length)
        m_b = jnp.minimum(l_b, n)
        contrib = jnp.abs(rv - 1.0) * m_b + (l_b - m_b)
        out_ref[0, 0] = jnp.sum(jnp.where(valid, contrib, 0.0)) * (1.0 / _B)


@jax.jit
def _tc_loss(alpha, ctc_log_probs, mask, text_length):
    out = pl.pallas_call(
        _tc_body,
        grid=(_B, _NCH),
        in_specs=[
            pl.BlockSpec(memory_space=pltpu.SMEM),
            pl.BlockSpec((_B, _T), lambda b, c: (0, 0)),
            pl.BlockSpec((1, _TCH, 128), lambda b, c: (b, c, 0)),
            pl.BlockSpec((_B, _T), lambda b, c: (0, 0)),
        ],
        out_specs=pl.BlockSpec(memory_space=pltpu.SMEM),
        out_shape=jax.ShapeDtypeStruct((1, 1), jnp.float32),
        scratch_shapes=[
            pltpu.VMEM((8, 128), jnp.float32),
            pltpu.VMEM((8, 128), jnp.float32),
        ],
    )(text_length, alpha, ctc_log_probs, mask)
    return out[0, 0]


def kernel(alpha, ctc_log_probs, mask, text_length):
    return _tc_loss(alpha, ctc_log_probs, mask, text_length)


# speedup vs baseline: 2.8221x; 2.8221x over previous
import math

import jax
import jax.numpy as jnp
from jax import lax
from jax.experimental import pallas as pl
from jax.experimental.pallas import tpu as pltpu

_SPIKE_THRESHOLD = math.log(0.9)
_B, _T, _V = 16, 2048, 512


def _tc_body(tl_ref, alpha_ref, ctc_ref, mask_ref, out_ref, cacc):
    b = pl.program_id(0)

    x = ctc_ref[0]
    trig = ((1.0 - x) > _SPIKE_THRESHOLD).astype(jnp.float32)
    m01 = (mask_ref[pl.ds(b, 1), :] != 0.0).astype(jnp.float32)
    y = jax.lax.dot_general(m01, trig, (((1,), (0,)), ((), ())),
                            preferred_element_type=jnp.float32)
    cacc[pl.ds(b, 1), :] = y

    @pl.when(b == _B - 1)
    def _final():
        lane = lax.broadcasted_iota(jnp.int32, (1, 128), 1)
        e0 = (lane == 0).astype(jnp.float32)
        counts = jax.lax.dot_general(
            e0, cacc[...], (((1,), (1,)), ((), ())),
            preferred_element_type=jnp.float32)
        ones_t = jnp.ones((1, _T), jnp.float32)
        rvs = jax.lax.dot_general(
            ones_t, alpha_ref[...], (((1,), (1,)), ((), ())),
            preferred_element_type=jnp.float32)
        lanes_b = lax.broadcasted_iota(jnp.int32, (1, _B), 1)
        lt = jnp.zeros((1, _B), jnp.float32)
        for i in range(_B):
            lt += jnp.where(lanes_b == i, tl_ref[i].astype(jnp.float32), 0.0)
        has = counts >= 1.0
        n = jnp.where(has, counts, 1.0)
        rv = jnp.where(has, rvs, 1.0)
        max_s = jnp.max(n)
        max_len = jnp.maximum(1.0, jnp.max(lt))
        length = jnp.minimum(max_s, max_len)
        l_b = jnp.minimum(lt, length)
        m_b = jnp.minimum(l_b, n)
        contrib = jnp.abs(rv - 1.0) * m_b + (l_b - m_b)
        out_ref[0, 0] = jnp.sum(contrib) * (1.0 / _B)


@jax.jit
def _tc_loss(alpha, ctc_log_probs, mask, text_length):
    out = pl.pallas_call(
        _tc_body,
        grid=(_B,),
        in_specs=[
            pl.BlockSpec(memory_space=pltpu.SMEM),
            pl.BlockSpec((_B, _T), lambda b: (0, 0)),
            pl.BlockSpec((1, _T, 128), lambda b: (b, 0, 0)),
            pl.BlockSpec((_B, _T), lambda b: (0, 0)),
        ],
        out_specs=pl.BlockSpec(memory_space=pltpu.SMEM),
        out_shape=jax.ShapeDtypeStruct((1, 1), jnp.float32),
        scratch_shapes=[
            pltpu.VMEM((_B, 128), jnp.float32),
        ],
    )(text_length, alpha, ctc_log_probs, mask)
    return out[0, 0]


def kernel(alpha, ctc_log_probs, mask, text_length):
    return _tc_loss(alpha, ctc_log_probs, mask, text_length)


# speedup vs baseline: 2.9296x; 1.0381x over previous
import math

import jax
import jax.numpy as jnp
import numpy as np
from jax import lax
from jax.experimental import pallas as pl
from jax.experimental.pallas import tpu as pltpu

_SPIKE_THRESHOLD = math.log(0.9)
_BLANK_CUT = float(np.float32(1.1053606))
_B, _T, _V = 16, 2048, 512


def _tc_body(tl_ref, alpha_ref, ctc_ref, mask_ref, out_ref, cacc):
    b = pl.program_id(0)

    x = ctc_ref[0]
    trig = (x < _BLANK_CUT).astype(jnp.bfloat16)
    m01 = (mask_ref[pl.ds(b, 1), :] != 0.0).astype(jnp.bfloat16)
    y = jax.lax.dot_general(m01, trig, (((1,), (0,)), ((), ())),
                            preferred_element_type=jnp.float32)
    cacc[pl.ds(b, 1), :] = y

    @pl.when(b == _B - 1)
    def _final():
        lane = lax.broadcasted_iota(jnp.int32, (1, 128), 1)
        e0 = (lane == 0).astype(jnp.float32)
        counts = jax.lax.dot_general(
            e0, cacc[...], (((1,), (1,)), ((), ())),
            preferred_element_type=jnp.float32)
        ones_t = jnp.ones((1, _T), jnp.float32)
        rvs = jax.lax.dot_general(
            ones_t, alpha_ref[...], (((1,), (1,)), ((), ())),
            preferred_element_type=jnp.float32)
        lanes_b = lax.broadcasted_iota(jnp.int32, (1, _B), 1)
        lt = jnp.zeros((1, _B), jnp.float32)
        for i in range(_B):
            lt += jnp.where(lanes_b == i, tl_ref[i].astype(jnp.float32), 0.0)
        has = counts >= 1.0
        n = jnp.where(has, counts, 1.0)
        rv = jnp.where(has, rvs, 1.0)
        max_s = jnp.max(n)
        max_len = jnp.maximum(1.0, jnp.max(lt))
        length = jnp.minimum(max_s, max_len)
        l_b = jnp.minimum(lt, length)
        m_b = jnp.minimum(l_b, n)
        contrib = jnp.abs(rv - 1.0) * m_b + (l_b - m_b)
        out_ref[0, 0] = jnp.sum(contrib) * (1.0 / _B)


@jax.jit
def _tc_loss(alpha, ctc_log_probs, mask, text_length):
    out = pl.pallas_call(
        _tc_body,
        grid=(_B,),
        in_specs=[
            pl.BlockSpec(memory_space=pltpu.SMEM),
            pl.BlockSpec((_B, _T), lambda b: (0, 0)),
            pl.BlockSpec((1, _T, 128), lambda b: (b, 0, 0)),
            pl.BlockSpec((_B, _T), lambda b: (0, 0)),
        ],
        out_specs=pl.BlockSpec(memory_space=pltpu.SMEM),
        out_shape=jax.ShapeDtypeStruct((1, 1), jnp.float32),
        scratch_shapes=[
            pltpu.VMEM((_B, 128), jnp.float32),
        ],
    )(text_length, alpha, ctc_log_probs, mask)
    return out[0, 0]


def kernel(alpha, ctc_log_probs, mask, text_length):
    return _tc_loss(alpha, ctc_log_probs, mask, text_length)


# speedup vs baseline: 4.1386x; 1.4127x over previous
import math

import jax
import jax.numpy as jnp
import numpy as np
from jax import lax
from jax.experimental import pallas as pl
from jax.experimental.pallas import tpu as pltpu

_SPIKE_THRESHOLD = math.log(0.9)
_BLANK_CUT = float(np.float32(1.1053606))
_B, _T, _V = 16, 2048, 512
_BPG = 2


def _tc_body(tl_ref, alpha_ref, ctc_ref, mask_ref, out_ref, cacc):
    g = pl.program_id(0)

    for j in range(_BPG):
        b = g * _BPG + j
        x = ctc_ref[j]
        trig = (x < _BLANK_CUT).astype(jnp.bfloat16)
        m01 = (mask_ref[pl.ds(b, 1), :] != 0.0).astype(jnp.bfloat16)
        y = jax.lax.dot_general(m01, trig, (((1,), (0,)), ((), ())),
                                preferred_element_type=jnp.float32)
        cacc[pl.ds(b, 1), :] = y

    @pl.when(g == _B // _BPG - 1)
    def _final():
        lane = lax.broadcasted_iota(jnp.int32, (1, 128), 1)
        e0 = (lane == 0).astype(jnp.float32)
        counts = jax.lax.dot_general(
            e0, cacc[...], (((1,), (1,)), ((), ())),
            preferred_element_type=jnp.float32)
        ones_t = jnp.ones((1, _T), jnp.float32)
        rvs = jax.lax.dot_general(
            ones_t, alpha_ref[...], (((1,), (1,)), ((), ())),
            preferred_element_type=jnp.float32)
        lanes_b = lax.broadcasted_iota(jnp.int32, (1, _B), 1)
        lt = jnp.zeros((1, _B), jnp.float32)
        for i in range(_B):
            lt += jnp.where(lanes_b == i, tl_ref[i].astype(jnp.float32), 0.0)
        has = counts >= 1.0
        n = jnp.where(has, counts, 1.0)
        rv = jnp.where(has, rvs, 1.0)
        max_s = jnp.max(n)
        max_len = jnp.maximum(1.0, jnp.max(lt))
        length = jnp.minimum(max_s, max_len)
        l_b = jnp.minimum(lt, length)
        m_b = jnp.minimum(l_b, n)
        contrib = jnp.abs(rv - 1.0) * m_b + (l_b - m_b)
        out_ref[0, 0] = jnp.sum(contrib) * (1.0 / _B)


@jax.jit
def _tc_loss(alpha, ctc_log_probs, mask, text_length):
    out = pl.pallas_call(
        _tc_body,
        grid=(_B // _BPG,),
        in_specs=[
            pl.BlockSpec(memory_space=pltpu.SMEM),
            pl.BlockSpec((_B, _T), lambda g: (0, 0)),
            pl.BlockSpec((_BPG, _T, 128), lambda g: (g, 0, 0)),
            pl.BlockSpec((_B, _T), lambda g: (0, 0)),
        ],
        out_specs=pl.BlockSpec(memory_space=pltpu.SMEM),
        out_shape=jax.ShapeDtypeStruct((1, 1), jnp.float32),
        scratch_shapes=[
            pltpu.VMEM((_B, 128), jnp.float32),
        ],
    )(text_length, alpha, ctc_log_probs, mask)
    return out[0, 0]


def kernel(alpha, ctc_log_probs, mask, text_length):
    return _tc_loss(alpha, ctc_log_probs, mask, text_length)


# speedup vs baseline: 5.1208x; 1.2373x over previous
import math

import jax
import jax.numpy as jnp
import numpy as np
from jax import lax
from jax.experimental import pallas as pl
from jax.experimental.pallas import tpu as pltpu

_SPIKE_THRESHOLD = math.log(0.9)
_BLANK_CUT = float(np.float32(1.1053606))
_B, _T, _V = 16, 2048, 512
_BPG = 4


def _tc_body(tl_ref, alpha_ref, ctc_ref, mask_ref, out_ref, cacc):
    g = pl.program_id(0)

    for j in range(_BPG):
        b = g * _BPG + j
        x = ctc_ref[j]
        trig = (x < _BLANK_CUT).astype(jnp.bfloat16)
        m01 = (mask_ref[pl.ds(b, 1), :] != 0.0).astype(jnp.bfloat16)
        y = jax.lax.dot_general(m01, trig, (((1,), (0,)), ((), ())),
                                preferred_element_type=jnp.float32)
        cacc[pl.ds(b, 1), :] = y

    @pl.when(g == _B // _BPG - 1)
    def _final():
        lane = lax.broadcasted_iota(jnp.int32, (1, 128), 1)
        e0 = (lane == 0).astype(jnp.float32)
        counts = jax.lax.dot_general(
            e0, cacc[...], (((1,), (1,)), ((), ())),
            preferred_element_type=jnp.float32)
        ones_t = jnp.ones((1, _T), jnp.float32)
        rvs = jax.lax.dot_general(
            ones_t, alpha_ref[...], (((1,), (1,)), ((), ())),
            preferred_element_type=jnp.float32)
        lanes_b = lax.broadcasted_iota(jnp.int32, (1, _B), 1)
        lt = jnp.zeros((1, _B), jnp.float32)
        for i in range(_B):
            lt += jnp.where(lanes_b == i, tl_ref[i].astype(jnp.float32), 0.0)
        has = counts >= 1.0
        n = jnp.where(has, counts, 1.0)
        rv = jnp.where(has, rvs, 1.0)
        max_s = jnp.max(n)
        max_len = jnp.maximum(1.0, jnp.max(lt))
        length = jnp.minimum(max_s, max_len)
        l_b = jnp.minimum(lt, length)
        m_b = jnp.minimum(l_b, n)
        contrib = jnp.abs(rv - 1.0) * m_b + (l_b - m_b)
        out_ref[0, 0] = jnp.sum(contrib) * (1.0 / _B)


@jax.jit
def _tc_loss(alpha, ctc_log_probs, mask, text_length):
    out = pl.pallas_call(
        _tc_body,
        grid=(_B // _BPG,),
        in_specs=[
            pl.BlockSpec(memory_space=pltpu.SMEM),
            pl.BlockSpec((_B, _T), lambda g: (0, 0)),
            pl.BlockSpec((_BPG, _T, 128), lambda g: (g, 0, 0)),
            pl.BlockSpec((_B, _T), lambda g: (0, 0)),
        ],
        out_specs=pl.BlockSpec(memory_space=pltpu.SMEM),
        out_shape=jax.ShapeDtypeStruct((1, 1), jnp.float32),
        scratch_shapes=[
            pltpu.VMEM((_B, 128), jnp.float32),
        ],
    )(text_length, alpha, ctc_log_probs, mask)
    return out[0, 0]


def kernel(alpha, ctc_log_probs, mask, text_length):
    return _tc_loss(alpha, ctc_log_probs, mask, text_length)
